# Initial kernel scaffold; baseline (speedup 1.0000x reference)
import jax
import jax.numpy as jnp
from jax.experimental import pallas as pl


def kernel(input_ids, token_table, ln_gamma, ln_beta):
    raise NotImplementedError("write your pallas kernel here")


# speedup vs baseline: 2.0680x; 2.0680x over previous
import functools
import math

import jax
import jax.numpy as jnp
import numpy as np
from jax import lax
from jax.experimental import pallas as pl
from jax.experimental.pallas import tpu as pltpu
from jax.experimental.pallas import tpu_sc as plsc

VOCAB = 1000000
D = 64
MAXTOK = 20
B = 1024
S = 50
EPS = 1e-5

N = B * S
NC, NS = 2, 16
NW = NC * NS
EV_PER_TILE = N // NW
CHUNK = 80
NCHUNK = EV_PER_TILE // CHUNK
L = 16
NBLK = CHUNK // L


def _pe_mean() -> np.ndarray:
    position = np.arange(MAXTOK, dtype=np.float64)[:, None]
    div_term = np.exp(
        np.arange(0, D, 2, dtype=np.float64) * (-math.log(10000.0) / D))
    pe = np.zeros((MAXTOK, D), dtype=np.float64)
    pe[:, 0::2] = np.sin(position * div_term)
    pe[:, 1::2] = np.cos(position * div_term)
    return pe.mean(axis=0).astype(np.float32)


def _rsqrt(x):
    i = plsc.bitcast(x, jnp.int32)
    i = jnp.int32(0x5F3759DF) - lax.shift_right_logical(i, 1)
    y = plsc.bitcast(i, jnp.float32)
    half = x * 0.5
    for _ in range(3):
        y = y * (1.5 - half * y * y)
    return y


def _sc_body(ids_h, tab_h, gam_h, bet_h, pem_h, out_h,
             idx_raw, idx_tok, acc, cnt0_v, row0_v, gam_v, bet_v, pem_v, sem):
    cid = lax.axis_index("c")
    sid = lax.axis_index("s")
    wid = sid * NC + cid
    base0 = wid * EV_PER_TILE

    pltpu.sync_copy(tab_h.at[0], row0_v)
    pltpu.sync_copy(gam_h, gam_v)
    pltpu.sync_copy(bet_h, bet_v)
    pltpu.sync_copy(pem_h, pem_v)

    iota = lax.iota(jnp.int32, L)
    zerov = jnp.zeros((L,), jnp.float32)
    inv_tok = jnp.float32(1.0 / MAXTOK)
    inv_d = jnp.float32(1.0 / D)

    def chunk_body(ci, _):
        base = base0 + ci * CHUNK
        pltpu.sync_copy(ids_h.at[pl.ds(base, CHUNK)], idx_raw)

        def t_body(t, cnts):
            tt = jnp.full((L,), t, jnp.int32)
            new = []
            for blk in range(NBLK):
                rows = iota + blk * L
                v = plsc.load_gather(idx_raw, [rows, tt])
                plsc.store_scatter(idx_tok, [tt, rows], v)
                new.append(cnts[blk] + jnp.where(v == 0, 1.0, 0.0))
            return tuple(new)

        cnts = lax.fori_loop(0, MAXTOK, t_body,
                             tuple(zerov for _ in range(NBLK)))
        for blk in range(NBLK):
            cnt0_v[pl.ds(blk * L, L)] = cnts[blk]

        def z_body(r, _):
            for j in range(D // L):
                acc[r, pl.ds(j * L, L)] = zerov
            return 0

        lax.fori_loop(0, CHUNK, z_body, 0)

        for t in range(MAXTOK):
            pltpu.async_copy(tab_h.at[idx_tok.at[t]], acc, sem, add=True)
        for t in range(MAXTOK):
            pltpu.make_async_copy(tab_h.at[idx_tok.at[t]], acc, sem).wait()

        cblk = [cnt0_v[pl.ds(blk * L, L)] for blk in range(NBLK)]

        def p1_body(d, carry):
            dd = jnp.full((L,), d, jnp.int32)
            r0 = plsc.load_gather(row0_v, [dd])
            pm = plsc.load_gather(pem_v, [dd])
            new1, new2 = [], []
            for blk in range(NBLK):
                rows = iota + blk * L
                c = plsc.load_gather(acc, [rows, dd])
                v = (c - cblk[blk] * r0) * inv_tok + pm
                new1.append(carry[blk] + v)
                new2.append(carry[NBLK + blk] + v * v)
            return tuple(new1) + tuple(new2)

        carry = lax.fori_loop(0, D, p1_body,
                              tuple(zerov for _ in range(2 * NBLK)))
        mus, rss = [], []
        for blk in range(NBLK):
            mu = carry[blk] * inv_d
            var = carry[NBLK + blk] * inv_d - mu * mu
            mus.append(mu)
            rss.append(_rsqrt(var + EPS))

        def p2_body(d, _):
            dd = jnp.full((L,), d, jnp.int32)
            r0 = plsc.load_gather(row0_v, [dd])
            pm = plsc.load_gather(pem_v, [dd])
            gg = plsc.load_gather(gam_v, [dd])
            bb = plsc.load_gather(bet_v, [dd])
            for blk in range(NBLK):
                rows = iota + blk * L
                c = plsc.load_gather(acc, [rows, dd])
                v = (c - cblk[blk] * r0) * inv_tok + pm
                o = (v - mus[blk]) * rss[blk] * gg + bb
                plsc.store_scatter(acc, [rows, dd], o)
            return 0

        lax.fori_loop(0, D, p2_body, 0)
        pltpu.sync_copy(acc, out_h.at[pl.ds(base, CHUNK)])
        return 0

    lax.fori_loop(0, NCHUNK, chunk_body, 0)


@jax.jit
def kernel(input_ids, token_table, ln_gamma, ln_beta):
    ids = input_ids.reshape(N, MAXTOK)
    pe_mean = jnp.asarray(_pe_mean())

    mesh = plsc.VectorSubcoreMesh(core_axis_name="c", subcore_axis_name="s",
                                  num_cores=NC, num_subcores=NS)
    run = pl.kernel(
        _sc_body,
        out_type=jax.ShapeDtypeStruct((N, D), jnp.float32),
        mesh=mesh,
        compiler_params=pltpu.CompilerParams(
            needs_layout_passes=False, use_tc_tiling_on_sc=False),
        scratch_types=[
            pltpu.VMEM((CHUNK, MAXTOK), jnp.int32),
            pltpu.VMEM((MAXTOK, CHUNK), jnp.int32),
            pltpu.VMEM((CHUNK, D), jnp.float32),
            pltpu.VMEM((CHUNK,), jnp.float32),
            pltpu.VMEM((D,), jnp.float32),
            pltpu.VMEM((D,), jnp.float32),
            pltpu.VMEM((D,), jnp.float32),
            pltpu.VMEM((D,), jnp.float32),
            pltpu.SemaphoreType.DMA,
        ],
    )
    out = run(ids, token_table, ln_gamma, ln_beta, pe_mean)
    return out.reshape(B, S, D)
